# baseline (device time: 101852 ns/iter reference)
import functools

import jax
import jax.numpy as jnp
from jax import lax
from jax.experimental import pallas as pl
from jax.experimental.pallas import tpu as pltpu

N_DEV = 8
M_BLK = 512
K = 4096
N = 8192
N_CHUNKS = 4
N_CHUNK = N // N_CHUNKS
N_SLOTS = 3


def kernel(x, w_mat):

    def body(x_ref, w_hbm, out_init, out_hbm, x_bf, comm_ref, w_buf, out_stage,
             send_sems, recv_sems, w_sems, out_sems):
        my = lax.axis_index("i")

        def out_copy(h):
            return pltpu.make_async_copy(
                out_stage.at[h % 2],
                out_hbm.at[:, pl.ds(h * N_CHUNK, N_CHUNK)],
                out_sems.at[h % 2],
            )

        def w_copy(t, slot):
            h, s = divmod(t, N_DEV)
            j = (my + s) % N_DEV
            return pltpu.make_async_copy(
                w_hbm.at[pl.ds(j * M_BLK, M_BLK),
                         pl.ds(h * N_CHUNK, N_CHUNK)],
                w_buf.at[slot],
                w_sems.at[slot],
            )

        for t in range(N_SLOTS - 1):
            w_copy(t, t).start()
        x_bf[...] = x_ref[...].astype(jnp.bfloat16)

        barrier = pltpu.get_barrier_semaphore()
        for r in range(1, N_DEV):
            pl.semaphore_signal(
                barrier, inc=1,
                device_id=((my + r) % N_DEV,),
                device_id_type=pl.DeviceIdType.MESH,
            )
        pl.semaphore_wait(barrier, N_DEV - 1)

        sends = []
        for r in range(1, N_DEV):
            d = (my - r) % N_DEV
            rdma = pltpu.make_async_remote_copy(
                src_ref=x_bf.at[pl.ds(d * M_BLK, M_BLK), :],
                dst_ref=comm_ref.at[r],
                send_sem=send_sems.at[r],
                recv_sem=recv_sems.at[r],
                device_id=(d,),
                device_id_type=pl.DeviceIdType.MESH,
            )
            rdma.start()
            sends.append(rdma)

        for h in range(N_CHUNKS):
            acc = None
            for s in range(N_DEV):
                t = h * N_DEV + s
                slot = t % N_SLOTS
                nxt = t + N_SLOTS - 1
                if nxt < N_CHUNKS * N_DEV:
                    w_copy(nxt, nxt % N_SLOTS).start()
                w_copy(t, slot).wait()
                if s == 0:
                    block = x_bf[pl.ds(my * M_BLK, M_BLK), :]
                else:
                    if h == 0:
                        recv = pltpu.make_async_remote_copy(
                            src_ref=comm_ref.at[s],
                            dst_ref=comm_ref.at[s],
                            send_sem=send_sems.at[s],
                            recv_sem=recv_sems.at[s],
                            device_id=((my + s) % N_DEV,),
                            device_id_type=pl.DeviceIdType.MESH,
                        )
                        recv.wait_recv()
                    block = comm_ref[s]
                part = jnp.dot(
                    block, w_buf[slot].astype(jnp.bfloat16),
                    preferred_element_type=jnp.float32,
                )
                acc = part if s == 0 else acc + part
            if h >= 2:
                out_copy(h - 2).wait()
            out_stage[h % 2] = acc
            out_copy(h).start()

        for h in range(max(N_CHUNKS - 2, 0), N_CHUNKS):
            out_copy(h).wait()
        for rdma in sends:
            rdma.wait_send()

        @functools.partial(pl.run_scoped,
                           sem2=pltpu.SemaphoreType.REGULAR)
        def _(sem2):
            for r in range(1, N_DEV):
                pl.semaphore_signal(
                    sem2, inc=1,
                    device_id=((my + r) % N_DEV,),
                    device_id_type=pl.DeviceIdType.MESH,
                )
            pl.semaphore_wait(sem2, N_DEV - 1)

    return pl.pallas_call(
        body,
        out_shape=jax.ShapeDtypeStruct((M_BLK, N), jnp.float32),
        in_specs=[
            pl.BlockSpec(memory_space=pltpu.VMEM),
            pl.BlockSpec(memory_space=pl.ANY),
            pl.BlockSpec(memory_space=pl.ANY),
        ],
        out_specs=pl.BlockSpec(memory_space=pl.ANY),
        input_output_aliases={2: 0},
        scratch_shapes=[
            pltpu.VMEM((K, M_BLK), jnp.bfloat16),
            pltpu.VMEM((N_DEV, M_BLK, M_BLK), jnp.bfloat16),
            pltpu.VMEM((N_SLOTS, M_BLK, N_CHUNK), jnp.float32),
            pltpu.VMEM((2, M_BLK, N_CHUNK), jnp.float32),
            pltpu.SemaphoreType.DMA((N_DEV,)),
            pltpu.SemaphoreType.DMA((N_DEV,)),
            pltpu.SemaphoreType.DMA((N_SLOTS,)),
            pltpu.SemaphoreType.DMA((2,)),
        ],
        compiler_params=pltpu.CompilerParams(
            collective_id=0,
            vmem_limit_bytes=63 * 1024 * 1024,
        ),
    )(x, w_mat, jnp.zeros((M_BLK, N), jnp.float32))


# device time: 95843 ns/iter; 1.0627x vs baseline; 1.0627x over previous
import functools

import jax
import jax.numpy as jnp
from jax import lax
from jax.experimental import pallas as pl
from jax.experimental.pallas import tpu as pltpu

N_DEV = 8
M_BLK = 512
K = 4096
N = 8192
N_CHUNKS = 4
N_CHUNK = N // N_CHUNKS
N_SLOTS = 3


def kernel(x, w_mat):

    def body(x_ref, w_hbm, out_hbm, x_bf, comm_ref, w_buf, out_stage,
             send_sems, recv_sems, w_sems, out_sems):
        my = lax.axis_index("i")

        def out_copy(h):
            return pltpu.make_async_copy(
                out_stage.at[h % 2],
                out_hbm.at[:, pl.ds(h * N_CHUNK, N_CHUNK)],
                out_sems.at[h % 2],
            )

        def w_copy(t, slot):
            h, s = divmod(t, N_DEV)
            j = (my + s) % N_DEV
            return pltpu.make_async_copy(
                w_hbm.at[pl.ds(j * M_BLK, M_BLK),
                         pl.ds(h * N_CHUNK, N_CHUNK)],
                w_buf.at[slot],
                w_sems.at[slot],
            )

        for t in range(N_SLOTS - 1):
            w_copy(t, t).start()
        x_bf[...] = x_ref[...].astype(jnp.bfloat16)

        barrier = pltpu.get_barrier_semaphore()
        for r in range(1, N_DEV):
            pl.semaphore_signal(
                barrier, inc=1,
                device_id=((my + r) % N_DEV,),
                device_id_type=pl.DeviceIdType.MESH,
            )
        pl.semaphore_wait(barrier, N_DEV - 1)

        sends = []
        for r in range(1, N_DEV):
            d = (my - r) % N_DEV
            rdma = pltpu.make_async_remote_copy(
                src_ref=x_bf.at[pl.ds(d * M_BLK, M_BLK), :],
                dst_ref=comm_ref.at[r],
                send_sem=send_sems.at[r],
                recv_sem=recv_sems.at[r],
                device_id=(d,),
                device_id_type=pl.DeviceIdType.MESH,
            )
            rdma.start()
            sends.append(rdma)

        for h in range(N_CHUNKS):
            acc = None
            for s in range(N_DEV):
                t = h * N_DEV + s
                slot = t % N_SLOTS
                nxt = t + N_SLOTS - 1
                if nxt < N_CHUNKS * N_DEV:
                    w_copy(nxt, nxt % N_SLOTS).start()
                w_copy(t, slot).wait()
                if s == 0:
                    block = x_bf[pl.ds(my * M_BLK, M_BLK), :]
                else:
                    if h == 0:
                        recv = pltpu.make_async_remote_copy(
                            src_ref=comm_ref.at[s],
                            dst_ref=comm_ref.at[s],
                            send_sem=send_sems.at[s],
                            recv_sem=recv_sems.at[s],
                            device_id=((my + s) % N_DEV,),
                            device_id_type=pl.DeviceIdType.MESH,
                        )
                        recv.wait_recv()
                    block = comm_ref[s]
                part = jnp.dot(
                    block, w_buf[slot].astype(jnp.bfloat16),
                    preferred_element_type=jnp.float32,
                )
                acc = part if s == 0 else acc + part
            if h >= 2:
                out_copy(h - 2).wait()
            out_stage[h % 2] = acc
            out_copy(h).start()

        for h in range(max(N_CHUNKS - 2, 0), N_CHUNKS):
            out_copy(h).wait()
        for rdma in sends:
            rdma.wait_send()

        @functools.partial(pl.run_scoped,
                           sem2=pltpu.SemaphoreType.REGULAR)
        def _(sem2):
            for r in range(1, N_DEV):
                pl.semaphore_signal(
                    sem2, inc=1,
                    device_id=((my + r) % N_DEV,),
                    device_id_type=pl.DeviceIdType.MESH,
                )
            pl.semaphore_wait(sem2, N_DEV - 1)

    return pl.pallas_call(
        body,
        out_shape=jax.ShapeDtypeStruct((M_BLK, N), jnp.float32),
        in_specs=[
            pl.BlockSpec(memory_space=pltpu.VMEM),
            pl.BlockSpec(memory_space=pl.ANY),
        ],
        out_specs=pl.BlockSpec(memory_space=pl.ANY),
        scratch_shapes=[
            pltpu.VMEM((K, M_BLK), jnp.bfloat16),
            pltpu.VMEM((N_DEV, M_BLK, M_BLK), jnp.bfloat16),
            pltpu.VMEM((N_SLOTS, M_BLK, N_CHUNK), jnp.float32),
            pltpu.VMEM((2, M_BLK, N_CHUNK), jnp.float32),
            pltpu.SemaphoreType.DMA((N_DEV,)),
            pltpu.SemaphoreType.DMA((N_DEV,)),
            pltpu.SemaphoreType.DMA((N_SLOTS,)),
            pltpu.SemaphoreType.DMA((2,)),
        ],
        compiler_params=pltpu.CompilerParams(
            collective_id=0,
            vmem_limit_bytes=63 * 1024 * 1024,
        ),
    )(x, w_mat)
